# baseline (speedup 1.0000x reference)
import functools

import jax
import jax.numpy as jnp
from jax import lax
from jax.experimental import pallas as pl
from jax.experimental.pallas import tpu as pltpu
from jax.experimental.pallas import tpu_sc as plsc

B, S, D = 4096, 200, 64
NUM_ROWS = 12
BS = B * S
D2 = 2 * D

NC, NS = 2, 16
NW = NC * NS
ROWS_PER_W = BS // NW
MB = 160
NMB = ROWS_PER_W // MB
G = MB // 16

_mesh = plsc.VectorSubcoreMesh(core_axis_name="c", subcore_axis_name="s")


@functools.partial(
    pl.kernel,
    out_type=jax.ShapeDtypeStruct((BS, D2), jnp.float32),
    mesh=_mesh,
    compiler_params=pltpu.CompilerParams(needs_layout_passes=False),
    scratch_types=[
        pltpu.VMEM((NUM_ROWS * D,), jnp.float32),
        pltpu.VMEM((MB,), jnp.int32),
        pltpu.VMEM((MB,), jnp.int32),
        pltpu.VMEM((MB, D), jnp.float32),
        pltpu.VMEM((MB, D), jnp.float32),
        pltpu.VMEM((MB, D2), jnp.float32),
        pltpu.VMEM((MB, D2), jnp.float32),
        pltpu.SemaphoreType.DMA,
        pltpu.SemaphoreType.DMA,
        pltpu.SemaphoreType.DMA,
        pltpu.SemaphoreType.DMA,
    ],
)
def _sc_interleave(past_hbm, rat_hbm, table_hbm, out_hbm,
                   table_v, rat_v0, rat_v1, past_v0, past_v1, out_v0, out_v1,
                   sem_in0, sem_in1, sem_out0, sem_out1):
    wid = lax.axis_index("s") * NC + lax.axis_index("c")
    wbase = wid * ROWS_PER_W

    pltpu.sync_copy(table_hbm, table_v)
    lane = lax.iota(jnp.int32, 16)

    bufs = ((rat_v0, past_v0, out_v0, sem_in0, sem_out0),
            (rat_v1, past_v1, out_v1, sem_in1, sem_out1))

    def issue_loads(m, bi):
        rat_v, past_v, _, sem_in, _ = bufs[bi]
        base = wbase + m * MB
        pltpu.async_copy(rat_hbm.at[pl.ds(base, MB)], rat_v, sem_in)
        pltpu.async_copy(past_hbm.at[pl.ds(base, MB)], past_v, sem_in)

    def wait_loads(m, bi):
        rat_v, past_v, _, sem_in, _ = bufs[bi]
        base = wbase + m * MB
        pltpu.make_async_copy(rat_hbm.at[pl.ds(base, MB)], rat_v, sem_in).wait()
        pltpu.make_async_copy(past_hbm.at[pl.ds(base, MB)], past_v, sem_in).wait()

    def issue_store(m, bi):
        _, _, out_v, _, sem_out = bufs[bi]
        base = wbase + m * MB
        pltpu.async_copy(out_v, out_hbm.at[pl.ds(base, MB)], sem_out)

    def drain_store(m, bi):
        _, _, out_v, _, sem_out = bufs[bi]
        base = wbase + m * MB
        pltpu.make_async_copy(out_v, out_hbm.at[pl.ds(base, MB)], sem_out).wait()

    def interleave(bi):
        rat_v, past_v, out_v, _, _ = bufs[bi]

        @plsc.parallel_loop(0, G, unroll=2)
        def group(g):
            rvec = rat_v[pl.ds(g * 16, 16)]
            tbase = rvec * D
            rows16 = lane + g * 16
            cbase = lane * 0 + D
            for w0 in range(0, D, 16):
                vals = [plsc.load_gather(table_v, [tbase + (w0 + k)])
                        for k in range(16)]
                for k in range(16):
                    plsc.store_scatter(out_v, [rows16, cbase + (w0 + k)], vals[k])
            for i in range(0, 16, 2):
                r0 = g * 16 + i
                r1 = r0 + 1
                p = [past_v[r0, pl.ds(w, 16)] for w in range(0, D, 16)]
                q = [past_v[r1, pl.ds(w, 16)] for w in range(0, D, 16)]
                for w in range(0, D, 16):
                    out_v[r0, pl.ds(w, 16)] = p[w // 16]
                    out_v[r1, pl.ds(w, 16)] = q[w // 16]

    issue_loads(0, 0)

    def body(i, _):
        m0 = 2 * i

        issue_loads(m0 + 1, 1)

        @pl.when(i >= 1)
        def _():
            drain_store(m0, 0)

        wait_loads(m0, 0)
        interleave(0)
        issue_store(m0, 0)

        @pl.when(i + 1 < NMB // 2)
        def _():
            issue_loads(m0 + 2, 0)

        @pl.when(i >= 1)
        def _():
            drain_store(m0 + 1, 1)

        wait_loads(m0 + 1, 1)
        interleave(1)
        issue_store(m0 + 1, 1)
        return 0

    lax.fori_loop(0, NMB // 2, body, 0)
    drain_store(NMB - 2, 0)
    drain_store(NMB - 1, 1)


def kernel(past_lengths, past_ids, past_embeddings, timestamps, ratings, rating_emb_weight):
    past2d = past_embeddings.reshape(BS, D)
    rat1d = ratings.reshape(BS)
    table1d = rating_emb_weight.reshape(NUM_ROWS * D)
    out = _sc_interleave(past2d, rat1d, table1d)
    return out.reshape(B, 2 * S, D)


# speedup vs baseline: 1.1154x; 1.1154x over previous
import functools

import jax
import jax.numpy as jnp
from jax import lax
from jax.experimental import pallas as pl
from jax.experimental.pallas import tpu as pltpu
from jax.experimental.pallas import tpu_sc as plsc

B, S, D = 4096, 200, 64
NUM_ROWS = 12
BS = B * S
D2 = 2 * D

NC, NS = 2, 16
NW = NC * NS
ROWS_PER_W = BS // NW
MB = 160
NMB = ROWS_PER_W // MB
G = MB // 16

_mesh = plsc.VectorSubcoreMesh(core_axis_name="c", subcore_axis_name="s")


@functools.partial(
    pl.kernel,
    out_type=jax.ShapeDtypeStruct((BS, D2), jnp.float32),
    mesh=_mesh,
    compiler_params=pltpu.CompilerParams(needs_layout_passes=False),
    scratch_types=[
        pltpu.VMEM((NUM_ROWS * D,), jnp.float32),
        pltpu.VMEM((MB,), jnp.int32),
        pltpu.VMEM((MB,), jnp.int32),
        pltpu.VMEM((MB, D), jnp.float32),
        pltpu.VMEM((MB, D), jnp.float32),
        pltpu.VMEM((MB, D2), jnp.float32),
        pltpu.VMEM((MB, D2), jnp.float32),
        pltpu.SemaphoreType.DMA,
        pltpu.SemaphoreType.DMA,
        pltpu.SemaphoreType.DMA,
        pltpu.SemaphoreType.DMA,
    ],
)
def _sc_interleave(past_hbm, rat_hbm, table_hbm, out_hbm,
                   table_v, rat_v0, rat_v1, past_v0, past_v1, out_v0, out_v1,
                   sem_in0, sem_in1, sem_out0, sem_out1):
    wid = lax.axis_index("s") * NC + lax.axis_index("c")
    wbase = wid * ROWS_PER_W

    pltpu.sync_copy(table_hbm, table_v)
    lane = lax.iota(jnp.int32, 16)

    bufs = ((rat_v0, past_v0, out_v0, sem_in0, sem_out0),
            (rat_v1, past_v1, out_v1, sem_in1, sem_out1))

    def issue_loads(m, bi):
        rat_v, past_v, _, sem_in, _ = bufs[bi]
        base = wbase + m * MB
        pltpu.async_copy(rat_hbm.at[pl.ds(base, MB)], rat_v, sem_in)
        pltpu.async_copy(past_hbm.at[pl.ds(base, MB)], past_v, sem_in)

    def wait_loads(m, bi):
        rat_v, past_v, _, sem_in, _ = bufs[bi]
        base = wbase + m * MB
        pltpu.make_async_copy(rat_hbm.at[pl.ds(base, MB)], rat_v, sem_in).wait()
        pltpu.make_async_copy(past_hbm.at[pl.ds(base, MB)], past_v, sem_in).wait()

    def issue_store(m, bi):
        _, _, out_v, _, sem_out = bufs[bi]
        base = wbase + m * MB
        pltpu.async_copy(out_v, out_hbm.at[pl.ds(base, MB)], sem_out)

    def drain_store(m, bi):
        _, _, out_v, _, sem_out = bufs[bi]
        base = wbase + m * MB
        pltpu.make_async_copy(out_v, out_hbm.at[pl.ds(base, MB)], sem_out).wait()

    def interleave(bi):
        rat_v, past_v, out_v, _, _ = bufs[bi]

        @plsc.parallel_loop(0, G, unroll=2)
        def group(g):
            rvec = rat_v[pl.ds(g * 16, 16)]
            tbase = rvec * D
            rows16 = lane + g * 16
            cbase = lane * 0 + D
            for w0 in range(0, D, 8):
                vals = [plsc.load_gather(table_v, [tbase + (w0 + k)])
                        for k in range(8)]
                for k in range(8):
                    plsc.store_scatter(out_v, [rows16, cbase + (w0 + k)], vals[k])
            for i in range(0, 16, 2):
                r0 = g * 16 + i
                r1 = r0 + 1
                p = [past_v[r0, pl.ds(w, 16)] for w in range(0, D, 16)]
                q = [past_v[r1, pl.ds(w, 16)] for w in range(0, D, 16)]
                for w in range(0, D, 16):
                    out_v[r0, pl.ds(w, 16)] = p[w // 16]
                    out_v[r1, pl.ds(w, 16)] = q[w // 16]

    issue_loads(0, 0)

    def body(i, _):
        m0 = 2 * i

        issue_loads(m0 + 1, 1)

        @pl.when(i >= 1)
        def _():
            drain_store(m0, 0)

        wait_loads(m0, 0)
        interleave(0)
        issue_store(m0, 0)

        @pl.when(i + 1 < NMB // 2)
        def _():
            issue_loads(m0 + 2, 0)

        @pl.when(i >= 1)
        def _():
            drain_store(m0 + 1, 1)

        wait_loads(m0 + 1, 1)
        interleave(1)
        issue_store(m0 + 1, 1)
        return 0

    lax.fori_loop(0, NMB // 2, body, 0)
    drain_store(NMB - 2, 0)
    drain_store(NMB - 1, 1)


def kernel(past_lengths, past_ids, past_embeddings, timestamps, ratings, rating_emb_weight):
    past2d = past_embeddings.reshape(BS, D)
    rat1d = ratings.reshape(BS)
    table1d = rating_emb_weight.reshape(NUM_ROWS * D)
    out = _sc_interleave(past2d, rat1d, table1d)
    return out.reshape(B, 2 * S, D)
